# speedup vs baseline: 1.1385x; 1.1385x over previous
import functools

import jax
import jax.numpy as jnp
from jax import lax
from jax.experimental import pallas as pl
from jax.experimental.pallas import tpu as pltpu
from jax.experimental.pallas import tpu_sc as plsc

VOCAB_SIZE = 100000
N_EMBED = 768
CONTEXT_SIZE = 2048
BATCH = 4
SEQ_LEN = 2048

NUM_CORES = 2
NUM_SUBCORES = 16
NUM_WORKERS = NUM_CORES * NUM_SUBCORES
S_PER_W = SEQ_LEN // NUM_WORKERS
LANES = 16
CHUNKS = N_EMBED // LANES


def _body(ids_hbm, wte_hbm, wpe_hbm, out_hbm, idx_v, wpe_v, rows_v, sem):
    wid = lax.axis_index("s") * NUM_CORES + lax.axis_index("c")
    s0 = wid * S_PER_W

    pltpu.sync_copy(wpe_hbm.at[pl.ds(s0, S_PER_W)], wpe_v)

    for b in range(BATCH):
        pltpu.sync_copy(ids_hbm.at[b, pl.ds(s0, S_PER_W)], idx_v)
        pltpu.async_copy(wte_hbm.at[idx_v], rows_v, sem).wait()

        def add_row(i, _):
            for j in range(CHUNKS):
                x = wpe_v[i, pl.ds(j * LANES, LANES)]
                plsc.addupdate(rows_v.at[i, pl.ds(j * LANES, LANES)], x)
            return 0

        lax.fori_loop(0, S_PER_W, add_row, 0)

        pltpu.sync_copy(rows_v, out_hbm.at[b, pl.ds(s0, S_PER_W)])


@jax.jit
def _embed(input_ids, wte, wpe):
    mesh = plsc.VectorSubcoreMesh(core_axis_name="c", subcore_axis_name="s")
    return pl.kernel(
        _body,
        out_type=jax.ShapeDtypeStruct((BATCH, SEQ_LEN, N_EMBED), jnp.float32),
        mesh=mesh,
        scratch_types=[
            pltpu.VMEM((S_PER_W,), jnp.int32),
            pltpu.VMEM((S_PER_W, N_EMBED), jnp.float32),
            pltpu.VMEM((S_PER_W, N_EMBED), jnp.float32),
            pltpu.SemaphoreType.DMA,
        ],
    )(input_ids, wte, wpe)


def kernel(input_ids, wte, wpe):
    return _embed(input_ids.astype(jnp.int32), wte, wpe)
